# baseline (speedup 1.0000x reference)
import functools

import jax
import jax.numpy as jnp
from jax import lax
from jax.experimental import pallas as pl
from jax.experimental.pallas import tpu as pltpu
from jax.experimental.pallas import tpu_sc as plsc


def kernel(W):
    mesh = plsc.VectorSubcoreMesh(core_axis_name="c", subcore_axis_name="s")

    @functools.partial(
        pl.kernel,
        mesh=mesh,
        out_type=jax.ShapeDtypeStruct((1, W.shape[1]), W.dtype),
        scratch_types=[pltpu.VMEM((1, W.shape[1]), W.dtype)],
    )
    def _lookup(w_hbm, out_hbm, row_v):
        first = (lax.axis_index("c") == 0) & (lax.axis_index("s") == 0)

        @pl.when(first)
        def _():
            pltpu.sync_copy(w_hbm.at[pl.ds(0, 1)], row_v)
            pltpu.sync_copy(row_v, out_hbm)

    return _lookup(W)
